# baseline (speedup 1.0000x reference)
import functools

import jax
import jax.numpy as jnp
from jax import lax
from jax.experimental import pallas as pl
from jax.experimental.pallas import tpu as pltpu
from jax.experimental.pallas import tpu_sc as plsc

B, P, X, Y = 1024, 64, 32, 32
M = 4096
F = P * X * Y

NC, NS, L = 2, 16, 16
NW = NC * NS
RPW = B // NW


def _policy_flatten_kernel(x_hbm, p_hbm, cx_hbm, cy_hbm, out_hbm,
                           flat_v, cx_v, cy_v, vals_v, gsem):
  wid = lax.axis_index("s") * NC + lax.axis_index("c")

  pltpu.sync_copy(p_hbm, flat_v)
  pltpu.sync_copy(cx_hbm, cx_v)
  pltpu.sync_copy(cy_hbm, cy_v)

  def fold(j, carry):
    sl = pl.ds(j * L, L)
    flat_v[sl] = flat_v[sl] * (X * Y) + cx_v[sl] * Y + cy_v[sl]
    return carry

  lax.fori_loop(0, M // L, fold, 0, unroll=4)

  def issue_row(i):
    b = wid * RPW + i
    buf = i % 2

    def issue(j, c):
      sl = pl.ds(j * L, L)
      gidx = flat_v[sl] + b * F
      pltpu.async_copy(x_hbm.at[gidx], vals_v.at[buf].at[sl], gsem)
      return c

    lax.fori_loop(0, M // L, issue, 0, unroll=8)

  def drain_row(i):
    pltpu.make_async_copy(x_hbm.at[pl.ds(0, M)], vals_v.at[i % 2],
                          gsem).wait()

  issue_row(0)

  def row(i, carry):
    b = wid * RPW + i

    @pl.when(i + 1 < RPW)
    def _():
      issue_row(i + 1)

    drain_row(i)
    pltpu.sync_copy(vals_v.at[i % 2], out_hbm.at[b])
    return carry

  lax.fori_loop(0, RPW, row, 0)


@jax.jit
def kernel(x, piece_orientation_indices, center_placement_x,
           center_placement_y):
  x1 = x.reshape(B * F)
  run = pl.kernel(
      _policy_flatten_kernel,
      out_type=jax.ShapeDtypeStruct((B, M), jnp.float32),
      mesh=plsc.VectorSubcoreMesh(core_axis_name="c", subcore_axis_name="s"),
      scratch_types=[
          pltpu.VMEM((M,), jnp.int32),
          pltpu.VMEM((M,), jnp.int32),
          pltpu.VMEM((M,), jnp.int32),
          pltpu.VMEM((2, M), jnp.float32),
          pltpu.SemaphoreType.DMA,
      ],
      compiler_params=pltpu.CompilerParams(needs_layout_passes=False),
  )
  return run(x1,
             piece_orientation_indices.astype(jnp.int32),
             center_placement_x.astype(jnp.int32),
             center_placement_y.astype(jnp.int32))


# speedup vs baseline: 2.8215x; 2.8215x over previous
import functools

import jax
import jax.numpy as jnp
from jax import lax
from jax.experimental import pallas as pl
from jax.experimental.pallas import tpu as pltpu
from jax.experimental.pallas import tpu_sc as plsc

B, P, X, Y = 1024, 64, 32, 32
M = 4096
F = P * X * Y

NC, NS, L = 2, 16, 16
NW = NC * NS
MT = M // NW
NBT = B // 128
NRUNS = B * F // 128


def _policy_flatten_kernel(x_hbm, p_hbm, cx_hbm, cy_hbm, out_hbm,
                           pv, cxv, cyv, idx_v, g_v, o_v, gsem, osem):
  wid = lax.axis_index("s") * NC + lax.axis_index("c")
  m0 = wid * MT

  pltpu.sync_copy(p_hbm.at[pl.ds(m0, MT)], pv)
  pltpu.sync_copy(cx_hbm.at[pl.ds(m0, MT)], cxv)
  pltpu.sync_copy(cy_hbm.at[pl.ds(m0, MT)], cyv)

  def fold(j, carry):
    sl = pl.ds(j * L, L)
    base = (pv[sl] * X + cxv[sl]) * 256 + cyv[sl] * 8
    for bt in range(NBT):
      idx_v[bt, sl] = base + bt
    return carry

  lax.fori_loop(0, MT // L, fold, 0)

  def start_gather(bt):
    return pltpu.async_copy(x_hbm.at[idx_v.at[bt]], g_v.at[bt % 2], gsem)

  def drain_gather(bt):
    pltpu.make_async_copy(x_hbm.at[pl.ds(0, MT)], g_v.at[bt % 2], gsem).wait()

  def out_slice(bt):
    return out_hbm.at[pl.ds(bt * 128, 128), pl.ds(m0, MT)]

  def drain_out(bt):
    pltpu.make_async_copy(o_v.at[bt % 2], out_slice(0), osem).wait()

  lane = lax.iota(jnp.int32, L)

  def transpose(bt):
    buf = bt % 2

    def per_b(bl, carry):
      col = jnp.full((L,), bl, jnp.int32)
      for j in range(MT // L):
        row = lane + (j * L)
        o_v[buf, bl, pl.ds(j * L, L)] = plsc.load_gather(
            g_v.at[buf], [row, col])
      return carry

    lax.fori_loop(0, 128, per_b, 0)

  start_gather(0)
  for bt in range(NBT):
    if bt + 1 < NBT:
      start_gather(bt + 1)
    drain_gather(bt)
    if bt >= 2:
      drain_out(bt)
    transpose(bt)
    pltpu.async_copy(o_v.at[bt % 2], out_slice(bt), osem)
  drain_out(0)
  drain_out(1)


@jax.jit
def kernel(x, piece_orientation_indices, center_placement_x,
           center_placement_y):
  xr = jnp.transpose(x, (1, 2, 3, 0)).reshape(NRUNS, 128)
  run = pl.kernel(
      _policy_flatten_kernel,
      out_type=jax.ShapeDtypeStruct((B, M), jnp.float32),
      mesh=plsc.VectorSubcoreMesh(core_axis_name="c", subcore_axis_name="s"),
      scratch_types=[
          pltpu.VMEM((MT,), jnp.int32),
          pltpu.VMEM((MT,), jnp.int32),
          pltpu.VMEM((MT,), jnp.int32),
          pltpu.VMEM((NBT, MT), jnp.int32),
          pltpu.VMEM((2, MT, 128), jnp.float32),
          pltpu.VMEM((2, 128, MT), jnp.float32),
          pltpu.SemaphoreType.DMA,
          pltpu.SemaphoreType.DMA,
      ],
      compiler_params=pltpu.CompilerParams(needs_layout_passes=False),
  )
  return run(xr,
             piece_orientation_indices.astype(jnp.int32),
             center_placement_x.astype(jnp.int32),
             center_placement_y.astype(jnp.int32))
